# Initial kernel scaffold; baseline (speedup 1.0000x reference)
import jax
import jax.numpy as jnp
from jax.experimental import pallas as pl


def kernel(x, edge_index, W, b, gamma, beta):
    raise NotImplementedError("write your pallas kernel here")


# speedup vs baseline: 10.6870x; 10.6870x over previous
import functools

import jax
import jax.numpy as jnp
from jax import lax
from jax.experimental import pallas as pl
from jax.experimental.pallas import tpu as pltpu
from jax.experimental.pallas import tpu_sc as plsc

_N = 10000
_E = 320000
_D = 128
_NP = 10240
_NC = 2
_NS = 16
_NW = _NC * _NS
_EPW = _E // _NW
_CH = 80
_NCH = _EPW // _CH
_RPT = _NP // _NS

_BLK = 1000
_G = _N // _BLK

@functools.lru_cache(maxsize=None)
def _sc_kernels():
    sc_mesh = plsc.VectorSubcoreMesh(core_axis_name="c", subcore_axis_name="s")

    @functools.partial(
        pl.kernel,
        out_type=jax.ShapeDtypeStruct((_NW, _NP), jnp.float32),
        mesh=sc_mesh,
        scratch_types=[
            pltpu.VMEM((_EPW,), jnp.int32),
            pltpu.VMEM((_NP,), jnp.float32),
        ],
        compiler_params=pltpu.CompilerParams(needs_layout_passes=False),
    )
    def deg_partials(dst_hbm, out_hbm, dst_v, hist_v):
        c = lax.axis_index("c")
        s = lax.axis_index("s")
        wid = s * _NC + c

        def zero_body(i, carry):
            hist_v[pl.ds(i * 16, 16)] = jnp.zeros((16,), jnp.float32)
            return carry

        lax.fori_loop(0, _NP // 16, zero_body, 0)

        pltpu.sync_copy(dst_hbm.at[pl.ds(wid * _EPW, _EPW)], dst_v)

        ones = jnp.ones((16,), jnp.float32)

        def hist_body(i, carry):
            idx = dst_v[pl.ds(i * 16, 16)]
            plsc.addupdate_scatter(hist_v, [idx], ones)
            return carry

        lax.fori_loop(0, _EPW // 16, hist_body, 0)

        pltpu.sync_copy(hist_v, out_hbm.at[wid])

    @functools.partial(
        pl.kernel,
        out_type=jax.ShapeDtypeStruct((_NC, _NP, _D), jnp.float32),
        mesh=sc_mesh,
        scratch_types=[
            pltpu.VMEM((_CH,), jnp.int32),
            pltpu.VMEM((_CH,), jnp.int32),
            pltpu.VMEM((_CH, _D), jnp.float32),
            pltpu.VMEM_SHARED((_NP, _D), jnp.float32),
            pltpu.SemaphoreType.DMA,
        ],
        compiler_params=pltpu.CompilerParams(needs_layout_passes=False),
    )
    def gather_segsum(src_hbm, dst_hbm, g_hbm, out_hbm, src_v, dst_v, rows_v,
                      acc_sh, sem):
        c = lax.axis_index("c")
        s = lax.axis_index("s")
        wid = s * _NC + c

        def zero_body(i, carry):
            for j in range(_D // 16):
                rows_v[i, pl.ds(j * 16, 16)] = jnp.zeros((16,), jnp.float32)
            return carry

        lax.fori_loop(0, _CH, zero_body, 0)
        row0 = s * _RPT
        for r in range(_RPT // _CH):
            pltpu.sync_copy(rows_v, acc_sh.at[pl.ds(row0 + r * _CH, _CH)])
        plsc.subcore_barrier()

        ebase = wid * _EPW

        def body(i, carry):
            off = ebase + i * _CH
            pltpu.sync_copy(src_hbm.at[pl.ds(off, _CH)], src_v)
            pltpu.sync_copy(dst_hbm.at[pl.ds(off, _CH)], dst_v)
            pltpu.async_copy(g_hbm.at[src_v], rows_v, sem).wait()
            pltpu.sync_copy(rows_v, acc_sh.at[dst_v], add=True)
            return carry

        lax.fori_loop(0, _NCH, body, 0)
        plsc.subcore_barrier()

        for r in range(_RPT // _CH):
            sl = pl.ds(row0 + r * _CH, _CH)
            pltpu.sync_copy(acc_sh.at[sl], out_hbm.at[c].at[sl])

    return deg_partials, gather_segsum


def _deg_partials(dst):
    return _sc_kernels()[0](dst)


def _gather_segsum(src, dst, g):
    return _sc_kernels()[1](src, dst, g)


def _isd_body(degp_ref, isd_ref):
    deg = jnp.sum(degp_ref[...], axis=0, keepdims=True)
    isd_ref[...] = jnp.where(deg > 0.0,
                             lax.rsqrt(jnp.maximum(deg, 1e-30)),
                             0.0)


def _stats_body(x_ref, o_ref, acc_ref):
    i = pl.program_id(0)

    @pl.when(i == 0)
    def _():
        acc_ref[...] = jnp.zeros_like(acc_ref)

    xb = x_ref[...]
    acc_ref[0:1, :] += jnp.sum(xb, axis=0, keepdims=True)
    acc_ref[1:2, :] += jnp.sum(xb * xb, axis=0, keepdims=True)

    @pl.when(i == pl.num_programs(0) - 1)
    def _():
        mean = acc_ref[0:1, :] * (1.0 / _N)
        var = acc_ref[1:2, :] * (1.0 / _N) - mean * mean
        rstd = lax.rsqrt(var + 1e-5)
        o_ref[...] = jnp.concatenate([mean, rstd], axis=0)


def _norm_body(x_ref, stats_ref, gamma_ref, beta_ref, isd_ref, h_ref, g_ref):
    mean = stats_ref[0:1, :]
    rstd = stats_ref[1:2, :]
    h = (x_ref[...] - mean) * rstd * gamma_ref[...] + beta_ref[...]
    h_ref[...] = h
    g_ref[...] = isd_ref[...] * h


def _combine_body(accp_ref, isd_ref, tx1_ref, g2_ref):
    accsum = accp_ref[0] + accp_ref[1]
    isd = isd_ref[...]
    tx1 = -isd * accsum
    tx1_ref[...] = tx1
    g2_ref[...] = isd * tx1


def _final_body(h_ref, tx1_ref, accp2_ref, isd_ref, w_ref, b_ref, o_ref):
    h = h_ref[...]
    tx1 = tx1_ref[...]
    acc2 = accp2_ref[0] + accp2_ref[1]
    tx2 = -2.0 * isd_ref[...] * acc2 - h
    out = jnp.dot(h, w_ref[0], preferred_element_type=jnp.float32)
    out += jnp.dot(tx1, w_ref[1], preferred_element_type=jnp.float32)
    out += jnp.dot(tx2, w_ref[2], preferred_element_type=jnp.float32)
    o_ref[...] = jnp.maximum(out + b_ref[...], 0.0)


def _isd_call(degp):
    return pl.pallas_call(
        _isd_body,
        out_shape=jax.ShapeDtypeStruct((1, _NP), jnp.float32),
    )(degp)


def _stats_call(x):
    return pl.pallas_call(
        _stats_body,
        grid=(_G,),
        in_specs=[pl.BlockSpec((_BLK, _D), lambda i: (i, 0))],
        out_specs=pl.BlockSpec((2, _D), lambda i: (0, 0)),
        out_shape=jax.ShapeDtypeStruct((2, _D), jnp.float32),
        scratch_shapes=[pltpu.VMEM((2, _D), jnp.float32)],
    )(x)


def _norm_call(x, stats, gamma, beta, isd_col):
    return pl.pallas_call(
        _norm_body,
        grid=(_G,),
        in_specs=[
            pl.BlockSpec((_BLK, _D), lambda i: (i, 0)),
            pl.BlockSpec((2, _D), lambda i: (0, 0)),
            pl.BlockSpec((1, _D), lambda i: (0, 0)),
            pl.BlockSpec((1, _D), lambda i: (0, 0)),
            pl.BlockSpec((_BLK, 1), lambda i: (i, 0)),
        ],
        out_specs=[
            pl.BlockSpec((_BLK, _D), lambda i: (i, 0)),
            pl.BlockSpec((_BLK, _D), lambda i: (i, 0)),
        ],
        out_shape=[
            jax.ShapeDtypeStruct((_N, _D), jnp.float32),
            jax.ShapeDtypeStruct((_N, _D), jnp.float32),
        ],
    )(x, stats, gamma, beta, isd_col)


def _combine_call(accp, isd_col):
    return pl.pallas_call(
        _combine_body,
        grid=(_G,),
        in_specs=[
            pl.BlockSpec((_NC, _BLK, _D), lambda i: (0, i, 0)),
            pl.BlockSpec((_BLK, 1), lambda i: (i, 0)),
        ],
        out_specs=[
            pl.BlockSpec((_BLK, _D), lambda i: (i, 0)),
            pl.BlockSpec((_BLK, _D), lambda i: (i, 0)),
        ],
        out_shape=[
            jax.ShapeDtypeStruct((_N, _D), jnp.float32),
            jax.ShapeDtypeStruct((_N, _D), jnp.float32),
        ],
    )(accp, isd_col)


def _final_call(h, tx1, accp2, isd_col, W, b_row):
    return pl.pallas_call(
        _final_body,
        grid=(_G,),
        in_specs=[
            pl.BlockSpec((_BLK, _D), lambda i: (i, 0)),
            pl.BlockSpec((_BLK, _D), lambda i: (i, 0)),
            pl.BlockSpec((_NC, _BLK, _D), lambda i: (0, i, 0)),
            pl.BlockSpec((_BLK, 1), lambda i: (i, 0)),
            pl.BlockSpec((3, _D, _D), lambda i: (0, 0, 0)),
            pl.BlockSpec((1, _D), lambda i: (0, 0)),
        ],
        out_specs=pl.BlockSpec((_BLK, _D), lambda i: (i, 0)),
        out_shape=jax.ShapeDtypeStruct((_N, _D), jnp.float32),
    )(h, tx1, accp2, isd_col, W, b_row)


def kernel(x, edge_index, W, b, gamma, beta):
    src = edge_index[0].astype(jnp.int32)
    dst = edge_index[1].astype(jnp.int32)

    degp = _deg_partials(dst)
    isd_row = _isd_call(degp)
    isd_col = isd_row.reshape(_NP, 1)[:_N]

    stats = _stats_call(x)
    h, g = _norm_call(x, stats, gamma.reshape(1, _D), beta.reshape(1, _D),
                      isd_col)

    accp1 = _gather_segsum(src, dst, g)
    tx1, g2 = _combine_call(accp1, isd_col)

    accp2 = _gather_segsum(src, dst, g2)
    return _final_call(h, tx1, accp2, isd_col, W, b.reshape(1, _D))
